# baseline (speedup 1.0000x reference)
import numpy as np
import jax
import jax.numpy as jnp
from jax import lax
from jax.experimental import pallas as pl
from jax.experimental.pallas import tpu as pltpu
from jax.experimental.pallas import tpu_sc as plsc

_D = 128
_T = 200
_B = 1024

_NC, _NS = 2, 16
_NW = _NC * _NS
_RPW = _B // _NW

_H0 = 104
_H1 = _T - _H0
_NBUF = 6
_NU = 2 * _RPW


def _pe_table():
    pos = np.arange(_T)[:, None].astype(np.float32)
    i = np.arange(0, _D, 2).astype(np.float32)
    denom = np.power(10000.0, i / _D)
    pe = np.zeros((_T, _D), dtype=np.float32)
    pe[:, 0::2] = np.sin(pos / denom)
    pe[:, 1::2] = np.cos(pos / denom)
    return pe


def _ulen(h):
    return _H0 if h == 0 else _H1


def _body(tok_hbm, table_hbm, pe_hbm, out_hbm,
          pe_v, idx_v, rows_v, gsems, wsems):
    c = lax.axis_index("c")
    s = lax.axis_index("s")
    wid = s * _NC + c
    base = wid * _RPW
    pltpu.sync_copy(pe_hbm, pe_v)
    pltpu.sync_copy(tok_hbm.at[pl.ds(base * _T, _RPW * _T)], idx_v)

    def gather_copy(r, h, buf):
        n = _ulen(h)
        cp = pltpu.make_async_copy(
            table_hbm.at[idx_v.at[pl.ds(r * _T + h * _H0, n)]],
            rows_v.at[buf, pl.ds(0, n)], gsems[buf])
        return cp

    def issue(r, h, buf, wait_wb):
        if wait_wb:
            n = _ulen(h)
            pltpu.make_async_copy(rows_v.at[buf, pl.ds(0, n)],
                                  out_hbm.at[0, pl.ds(0, n)],
                                  wsems[buf]).wait()
        gather_copy(r, h, buf).start()

    def consume(r, h, buf):
        n = _ulen(h)
        gather_copy(r, h, buf).wait()

        def add_row(i, inner):
            for j in range(_D // 16):
                sl = pl.ds(j * 16, 16)
                rows_v[buf, i, sl] = rows_v[buf, i, sl] + pe_v[h * _H0 + i, sl]
            return inner

        lax.fori_loop(0, n, add_row, 0)
        pltpu.make_async_copy(rows_v.at[buf, pl.ds(0, n)],
                              out_hbm.at[base + r, pl.ds(h * _H0, n)],
                              wsems[buf]).start()

    issue(0, 0, 0, False)
    issue(0, 1, 1, False)
    issue(1, 0, 2, False)
    issue(1, 1, 3, False)
    issue(2, 0, 4, False)
    consume(0, 0, 0)
    issue(2, 1, 5, False)
    consume(0, 1, 1)

    def pipe(k, carry):
        for cc in range(_NBUF):
            issue(3 * k + 3 + (cc // 2), cc % 2, cc, True)
            consume(3 * k + ((2 + cc) // 2), cc % 2, (2 + cc) % 6)
        return carry

    lax.fori_loop(0, 9, pipe, 0)

    issue(30, 0, 0, True)
    consume(28, 0, 2)
    issue(30, 1, 1, True)
    consume(28, 1, 3)
    issue(31, 0, 2, True)
    consume(29, 0, 4)
    issue(31, 1, 3, True)
    consume(29, 1, 5)
    consume(30, 0, 0)
    consume(30, 1, 1)
    consume(31, 0, 2)
    consume(31, 1, 3)
    for buf in range(_NBUF):
        n = _ulen(buf % 2)
        pltpu.make_async_copy(rows_v.at[buf, pl.ds(0, n)],
                              out_hbm.at[0, pl.ds(0, n)],
                              wsems[buf]).wait()


def kernel(token_ids, table, StartToken, EndToken):
    tok = token_ids.astype(jnp.int32).reshape(-1)
    pe = jnp.asarray(_pe_table())
    mesh = plsc.VectorSubcoreMesh(core_axis_name="c", subcore_axis_name="s")
    k = pl.kernel(
        _body,
        mesh=mesh,
        out_type=jax.ShapeDtypeStruct((_B, _T, _D), jnp.float32),
        scratch_types=[
            pltpu.VMEM((_T, _D), jnp.float32),
            pltpu.VMEM((_RPW * _T,), jnp.int32),
            pltpu.VMEM((_NBUF, _H0, _D), jnp.float32),
            [pltpu.SemaphoreType.DMA] * _NBUF,
            [pltpu.SemaphoreType.DMA] * _NBUF,
        ],
    )
    return k(tok, table, pe)


# speedup vs baseline: 1.0176x; 1.0176x over previous
import numpy as np
import jax
import jax.numpy as jnp
from jax import lax
from jax.experimental import pallas as pl
from jax.experimental.pallas import tpu as pltpu
from jax.experimental.pallas import tpu_sc as plsc

_D = 128
_T = 200
_B = 1024

_NC, _NS = 2, 16
_NW = _NC * _NS
_RPW = _B // _NW

_H0 = 104
_H1 = _T - _H0
_NBUF = 6
_NU = 2 * _RPW


def _pe_table():
    pos = np.arange(_T)[:, None].astype(np.float32)
    i = np.arange(0, _D, 2).astype(np.float32)
    denom = np.power(10000.0, i / _D)
    pe = np.zeros((_T, _D), dtype=np.float32)
    pe[:, 0::2] = np.sin(pos / denom)
    pe[:, 1::2] = np.cos(pos / denom)
    return pe


def _ulen(h):
    return _H0 if h == 0 else _H1


def _body(tok_hbm, table_hbm, pe_hbm, out_hbm,
          pe_v, idx_v, rows_v, gsems, wsems, psem):
    c = lax.axis_index("c")
    s = lax.axis_index("s")
    wid = s * _NC + c
    base = wid * _RPW
    pe_cp = pltpu.make_async_copy(pe_hbm, pe_v, psem)
    pe_cp.start()
    pltpu.sync_copy(tok_hbm.at[pl.ds(base * _T, _RPW * _T)], idx_v)

    def gather_copy(r, h, buf):
        n = _ulen(h)
        cp = pltpu.make_async_copy(
            table_hbm.at[idx_v.at[pl.ds(r * _T + h * _H0, n)]],
            rows_v.at[buf, pl.ds(0, n)], gsems[buf])
        return cp

    def issue(r, h, buf, wait_wb):
        if wait_wb:
            n = _ulen(h)
            pltpu.make_async_copy(rows_v.at[buf, pl.ds(0, n)],
                                  out_hbm.at[0, pl.ds(0, n)],
                                  wsems[buf]).wait()
        gather_copy(r, h, buf).start()

    def consume(r, h, buf):
        n = _ulen(h)
        gather_copy(r, h, buf).wait()

        def add_row(i, inner):
            for j in range(_D // 16):
                sl = pl.ds(j * 16, 16)
                rows_v[buf, i, sl] = rows_v[buf, i, sl] + pe_v[h * _H0 + i, sl]
            return inner

        lax.fori_loop(0, n, add_row, 0)
        pltpu.make_async_copy(rows_v.at[buf, pl.ds(0, n)],
                              out_hbm.at[base + r, pl.ds(h * _H0, n)],
                              wsems[buf]).start()

    issue(0, 0, 0, False)
    issue(0, 1, 1, False)
    issue(1, 0, 2, False)
    issue(1, 1, 3, False)
    issue(2, 0, 4, False)
    pe_cp.wait()
    consume(0, 0, 0)
    issue(2, 1, 5, False)
    consume(0, 1, 1)

    def pipe(k, carry):
        for cc in range(_NBUF):
            issue(3 * k + 3 + (cc // 2), cc % 2, cc, True)
            consume(3 * k + ((2 + cc) // 2), cc % 2, (2 + cc) % 6)
        return carry

    lax.fori_loop(0, 9, pipe, 0)

    issue(30, 0, 0, True)
    consume(28, 0, 2)
    issue(30, 1, 1, True)
    consume(28, 1, 3)
    issue(31, 0, 2, True)
    consume(29, 0, 4)
    issue(31, 1, 3, True)
    consume(29, 1, 5)
    consume(30, 0, 0)
    consume(30, 1, 1)
    consume(31, 0, 2)
    consume(31, 1, 3)
    for buf in range(_NBUF):
        n = _ulen(buf % 2)
        pltpu.make_async_copy(rows_v.at[buf, pl.ds(0, n)],
                              out_hbm.at[0, pl.ds(0, n)],
                              wsems[buf]).wait()


def kernel(token_ids, table, StartToken, EndToken):
    tok = token_ids.astype(jnp.int32).reshape(-1)
    pe = jnp.asarray(_pe_table())
    mesh = plsc.VectorSubcoreMesh(core_axis_name="c", subcore_axis_name="s")
    k = pl.kernel(
        _body,
        mesh=mesh,
        out_type=jax.ShapeDtypeStruct((_B, _T, _D), jnp.float32),
        scratch_types=[
            pltpu.VMEM((_T, _D), jnp.float32),
            pltpu.VMEM((_RPW * _T,), jnp.int32),
            pltpu.VMEM((_NBUF, _H0, _D), jnp.float32),
            [pltpu.SemaphoreType.DMA] * _NBUF,
            [pltpu.SemaphoreType.DMA] * _NBUF,
            pltpu.SemaphoreType.DMA,
        ],
    )
    return k(tok, table, pe)


# speedup vs baseline: 1.0184x; 1.0008x over previous
import numpy as np
import jax
import jax.numpy as jnp
from jax import lax
from jax.experimental import pallas as pl
from jax.experimental.pallas import tpu as pltpu
from jax.experimental.pallas import tpu_sc as plsc

_D = 128
_T = 200
_B = 1024

_NC, _NS = 2, 16
_NW = _NC * _NS
_RPW = _B // _NW

_H0 = 104
_H1 = _T - _H0
_NBUF = 6


def _pe_table():
    pos = np.arange(_T)[:, None].astype(np.float32)
    i = np.arange(0, _D, 2).astype(np.float32)
    denom = np.power(10000.0, i / _D)
    pe = np.zeros((_T, _D), dtype=np.float32)
    pe[:, 0::2] = np.sin(pos / denom)
    pe[:, 1::2] = np.cos(pos / denom)
    return pe


def _ulen(h):
    return _H0 if h == 0 else _H1


def _body(tok_hbm, table_hbm, pe_hbm, out_hbm,
          pe_v, idx_v, rows_v, gsems, wsems, psem):
    c = lax.axis_index("c")
    s = lax.axis_index("s")
    wid = s * _NC + c
    base = wid * _RPW
    pe_cp = pltpu.make_async_copy(pe_hbm, pe_v, psem)
    pe_cp.start()
    pltpu.sync_copy(tok_hbm.at[pl.ds(base * _T, _RPW * _T)], idx_v)

    def gather_copy(r, h, buf):
        n = _ulen(h)
        cp = pltpu.make_async_copy(
            table_hbm.at[idx_v.at[pl.ds(r * _T + h * _H0, n)]],
            rows_v.at[buf, pl.ds(0, n)], gsems[buf])
        return cp

    def issue(r, h, buf, wait_wb):
        if wait_wb:
            n = _ulen(h)
            pltpu.make_async_copy(rows_v.at[buf, pl.ds(0, n)],
                                  out_hbm.at[0, pl.ds(0, n)],
                                  wsems[buf]).wait()
        gather_copy(r, h, buf).start()

    def consume(r, h, buf):
        n = _ulen(h)
        gather_copy(r, h, buf).wait()

        def add_row(i, inner):
            for j in range(_D // 16):
                sl = pl.ds(j * 16, 16)
                rows_v[buf, i, sl] = rows_v[buf, i, sl] + pe_v[h * _H0 + i, sl]
            return inner

        lax.fori_loop(0, n, add_row, 0)
        pltpu.make_async_copy(rows_v.at[buf, pl.ds(0, n)],
                              out_hbm.at[base + r, pl.ds(h * _H0, n)],
                              wsems[buf]).start()

    issue(0, 0, 0, False)
    issue(0, 1, 1, False)
    issue(1, 0, 2, False)
    issue(1, 1, 3, False)
    issue(2, 0, 4, False)
    pe_cp.wait()
    consume(0, 0, 0)
    issue(2, 1, 5, False)
    consume(0, 1, 1)

    def pipe(k, carry):
        for cc in range(_NBUF):
            issue(3 * k + 3 + (cc // 2), cc % 2, cc, True)
            consume(3 * k + ((2 + cc) // 2), cc % 2, (2 + cc) % 6)
        return carry

    lax.fori_loop(0, 9, pipe, 0)

    issue(30, 0, 0, True)
    consume(28, 0, 2)
    issue(30, 1, 1, True)
    consume(28, 1, 3)
    issue(31, 0, 2, True)
    consume(29, 0, 4)
    issue(31, 1, 3, True)
    consume(29, 1, 5)
    consume(30, 0, 0)
    consume(30, 1, 1)
    consume(31, 0, 2)
    consume(31, 1, 3)
    for buf in range(_NBUF):
        n = _ulen(buf % 2)
        pltpu.make_async_copy(rows_v.at[buf, pl.ds(0, n)],
                              out_hbm.at[0, pl.ds(0, n)],
                              wsems[buf]).wait()


def kernel(token_ids, table, StartToken, EndToken):
    tok = token_ids.astype(jnp.int32).reshape(-1)
    pe = jnp.asarray(_pe_table())
    mesh = plsc.VectorSubcoreMesh(core_axis_name="c", subcore_axis_name="s")
    k = pl.kernel(
        _body,
        mesh=mesh,
        out_type=jax.ShapeDtypeStruct((_B, _T, _D), jnp.float32),
        scratch_types=[
            pltpu.VMEM((_T, _D), jnp.float32),
            pltpu.VMEM((_RPW * _T,), jnp.int32),
            pltpu.VMEM((_NBUF, _H0, _D), jnp.float32),
            [pltpu.SemaphoreType.DMA] * _NBUF,
            [pltpu.SemaphoreType.DMA] * _NBUF,
            pltpu.SemaphoreType.DMA,
        ],
    )
    return k(tok, table, pe)
